# baseline (speedup 1.0000x reference)
import functools

import jax
import jax.numpy as jnp
from jax import lax
from jax.experimental import pallas as pl
from jax.experimental.pallas import tpu as pltpu
from jax.experimental.pallas import tpu_sc as plsc

N = 10000
E = 320000
D = 128
NC = 2
NS = 16
NW = NC * NS
EPW = E // NW
CHUNK = 128
FULL_CHUNKS = EPW // CHUNK
TAIL = EPW - FULL_CHUNKS * CHUNK
RPS = 624
ZROWS = 16
LAST = NS * RPS
LREM = N - LAST
DEGW = 128
BR = 1000


def _vmesh():
    return plsc.VectorSubcoreMesh(core_axis_name="c", subcore_axis_name="s")


def _sc_degree(dst):
    @functools.partial(
        pl.kernel,
        out_type=jax.ShapeDtypeStruct((NC, N, DEGW), jnp.float32),
        mesh=_vmesh(),
        scratch_types=[
            pltpu.VMEM((CHUNK,), jnp.int32),
            pltpu.VMEM((CHUNK,), jnp.int32),
            pltpu.VMEM((TAIL,), jnp.int32),
            pltpu.VMEM((CHUNK, DEGW), jnp.float32),
            pltpu.VMEM((ZROWS, DEGW), jnp.float32),
            pltpu.VMEM_SHARED((N, DEGW), jnp.float32),
            pltpu.SemaphoreType.DMA,
            pltpu.SemaphoreType.DMA,
            pltpu.SemaphoreType.DMA,
            pltpu.SemaphoreType.DMA,
        ],
    )
    def deg_kernel(dst_hbm, out_hbm, idxA, idxB, idxt_v, ones_v, z_v, deg_sh,
                   s_iA, s_iB, s_sA, s_sB):
        cid = lax.axis_index("c")
        sid = lax.axis_index("s")
        wid = cid * NS + sid
        one16 = jnp.ones((16,), jnp.float32)
        zero16 = jnp.zeros((16,), jnp.float32)

        @pl.loop(0, CHUNK)
        def _(i):
            @pl.loop(0, DEGW // 16)
            def _(j):
                ones_v[i, pl.ds(j * 16, 16)] = one16

        @pl.loop(0, ZROWS)
        def _(i):
            @pl.loop(0, DEGW // 16)
            def _(j):
                z_v[i, pl.ds(j * 16, 16)] = zero16

        @pl.loop(0, RPS // ZROWS)
        def _(k):
            pltpu.sync_copy(z_v, deg_sh.at[pl.ds(sid * RPS + k * ZROWS, ZROWS)])

        @pl.when(sid == NS - 1)
        def _():
            pltpu.sync_copy(z_v.at[pl.ds(0, LREM)], deg_sh.at[pl.ds(LAST, LREM)])

        plsc.subcore_barrier()
        base = wid * EPW

        def idx_start(c, idx, sem):
            pltpu.async_copy(dst_hbm.at[pl.ds(base + c * CHUNK, CHUNK)], idx, sem)

        def idx_wait(idx, sem):
            pltpu.make_async_copy(dst_hbm.at[pl.ds(base, CHUNK)], idx, sem).wait()

        def scat_start(idx, sem):
            pltpu.async_copy(ones_v, deg_sh.at[idx], sem, add=True)

        def scat_wait(idx, sem):
            pltpu.make_async_copy(ones_v, deg_sh.at[idx], sem).wait()

        idx_start(0, idxA, s_iA)
        idx_wait(idxA, s_iA)
        scat_start(idxA, s_sA)
        idx_start(1, idxB, s_iB)

        @pl.loop(0, (FULL_CHUNKS - 2) // 2)
        def _(k):
            idx_wait(idxB, s_iB)
            scat_wait(idxA, s_sA)
            scat_start(idxB, s_sB)
            idx_start(2 * k + 2, idxA, s_iA)
            idx_wait(idxA, s_iA)
            scat_wait(idxB, s_sB)
            scat_start(idxA, s_sA)

            @pl.when(k < (FULL_CHUNKS - 2) // 2 - 1)
            def _():
                idx_start(2 * k + 3, idxB, s_iB)

        scat_wait(idxA, s_sA)
        idx_start(FULL_CHUNKS - 1, idxB, s_iB)
        idx_wait(idxB, s_iB)
        scat_start(idxB, s_sB)
        scat_wait(idxB, s_sB)

        pltpu.sync_copy(dst_hbm.at[pl.ds(base + FULL_CHUNKS * CHUNK, TAIL)], idxt_v)
        pltpu.sync_copy(ones_v.at[pl.ds(0, TAIL)], deg_sh.at[idxt_v], add=True)

        plsc.subcore_barrier()
        pltpu.sync_copy(deg_sh.at[pl.ds(sid * RPS, RPS)],
                        out_hbm.at[cid].at[pl.ds(sid * RPS, RPS)])

        @pl.when(sid == NS - 1)
        def _():
            pltpu.sync_copy(deg_sh.at[pl.ds(LAST, LREM)],
                            out_hbm.at[cid].at[pl.ds(LAST, LREM)])

    return deg_kernel(dst)


def _sc_aggregate(g, src, dst):
    ACH = 64
    AFC = (EPW - TAIL) // ACH
    NB = 4
    NIT = (AFC - 3) // NB
    REM = (AFC - 3) - NB * NIT

    @functools.partial(
        pl.kernel,
        out_type=jax.ShapeDtypeStruct((NC, N, D), jnp.float32),
        mesh=_vmesh(),
        scratch_types=(
            [pltpu.VMEM((ACH,), jnp.int32)] * (2 * NB)
            + [pltpu.VMEM((TAIL,), jnp.int32)] * 2
            + [pltpu.VMEM((ACH, D), jnp.float32)] * NB
            + [pltpu.VMEM((TAIL, D), jnp.float32)]
            + [pltpu.VMEM((ZROWS, D), jnp.float32)]
            + [pltpu.VMEM_SHARED((N, D), jnp.float32)]
            + [pltpu.SemaphoreType.DMA] * (3 * NB)
        ),
    )
    def agg_kernel(g_hbm, src_hbm, dst_hbm, out_hbm,
                   si0, di0, si1, di1, si2, di2, si3, di3, sit_v, dit_v,
                   rows0, rows1, rows2, rows3, rowst_v, z_v, s_sh,
                   ssi0, sdi0, sg0, ssi1, sdi1, sg1,
                   ssi2, sdi2, sg2, ssi3, sdi3, sg3):
        cid = lax.axis_index("c")
        sid = lax.axis_index("s")
        wid = cid * NS + sid
        zero16 = jnp.zeros((16,), jnp.float32)

        bufs = [(si0, di0, rows0, ssi0, sdi0, sg0),
                (si1, di1, rows1, ssi1, sdi1, sg1),
                (si2, di2, rows2, ssi2, sdi2, sg2),
                (si3, di3, rows3, ssi3, sdi3, sg3)]

        @pl.loop(0, ZROWS)
        def _(i):
            @pl.loop(0, D // 16)
            def _(j):
                z_v[i, pl.ds(j * 16, 16)] = zero16

        @pl.loop(0, RPS // ZROWS)
        def _(k):
            pltpu.sync_copy(z_v, s_sh.at[pl.ds(sid * RPS + k * ZROWS, ZROWS)])

        @pl.when(sid == NS - 1)
        def _():
            pltpu.sync_copy(z_v.at[pl.ds(0, LREM)], s_sh.at[pl.ds(LAST, LREM)])

        plsc.subcore_barrier()
        base = wid * EPW

        def idx_load(c, b):
            off = base + c * ACH
            pltpu.async_copy(src_hbm.at[pl.ds(off, ACH)], b[0], b[3])
            pltpu.async_copy(dst_hbm.at[pl.ds(off, ACH)], b[1], b[4])
            pltpu.make_async_copy(src_hbm.at[pl.ds(base, ACH)], b[0], b[3]).wait()
            pltpu.make_async_copy(dst_hbm.at[pl.ds(base, ACH)], b[1], b[4]).wait()

        def gath_start(b):
            pltpu.async_copy(g_hbm.at[b[0]], b[2], b[5])

        def gath_wait(b):
            pltpu.make_async_copy(g_hbm.at[b[0]], b[2], b[5]).wait()

        def scat_start(b):
            pltpu.async_copy(b[2], s_sh.at[b[1]], b[5], add=True)

        def scat_wait(b):
            pltpu.make_async_copy(b[2], s_sh.at[b[1]], b[5]).wait()

        def step(c, X, W, start_next=True):
            gath_wait(X)
            scat_start(X)
            scat_wait(W)
            if start_next:
                idx_load(c + 3, W)
                gath_start(W)

        idx_load(0, bufs[0])
        gath_start(bufs[0])
        idx_load(1, bufs[1])
        gath_start(bufs[1])
        idx_load(2, bufs[2])
        gath_start(bufs[2])
        gath_wait(bufs[0])
        scat_start(bufs[0])
        idx_load(3, bufs[3])
        gath_start(bufs[3])
        step(1, bufs[1], bufs[0])
        step(2, bufs[2], bufs[1])

        @pl.loop(0, NIT)
        def _(k):
            for p in range(NB):
                c = NB * k + 3 + p
                X = bufs[(3 + p) % NB]
                W = bufs[(2 + p) % NB]
                if p <= 1:
                    step(c, X, W)
                else:
                    @pl.when(k < NIT - 1)
                    def _(c=c, X=X, W=W):
                        step(c, X, W)

                    @pl.when(k == NIT - 1)
                    def _(c=c, X=X, W=W):
                        step(c, X, W, start_next=False)

        cl = AFC - 1
        step(cl, bufs[cl % NB], bufs[(cl - 1) % NB], start_next=False)
        scat_wait(bufs[cl % NB])

        tbase = base + AFC * ACH
        pltpu.sync_copy(src_hbm.at[pl.ds(tbase, TAIL)], sit_v)
        pltpu.sync_copy(dst_hbm.at[pl.ds(tbase, TAIL)], dit_v)
        pltpu.sync_copy(g_hbm.at[sit_v], rowst_v)
        pltpu.sync_copy(rowst_v, s_sh.at[dit_v], add=True)

        plsc.subcore_barrier()
        pltpu.sync_copy(s_sh.at[pl.ds(sid * RPS, RPS)],
                        out_hbm.at[cid].at[pl.ds(sid * RPS, RPS)])

        @pl.when(sid == NS - 1)
        def _():
            pltpu.sync_copy(s_sh.at[pl.ds(LAST, LREM)],
                            out_hbm.at[cid].at[pl.ds(LAST, LREM)])

    return agg_kernel(g, src, dst)


def _dot(a, b):
    return lax.dot_general(a, b, (((1,), (0,)), ((), ())),
                           preferred_element_type=jnp.float32,
                           precision=lax.Precision.HIGHEST)


def _tc_mm_scale(deg_parts, x, w):
    def body(dp_ref, x_ref, w_ref, g_ref, dis_ref):
        deg = dp_ref[0, :, 0:1] + dp_ref[1, :, 0:1] + 1.0
        dis = lax.rsqrt(deg)
        g_ref[...] = _dot(x_ref[...], w_ref[...]) * dis
        dis_ref[...] = dis

    return pl.pallas_call(
        body,
        grid=(N // BR,),
        in_specs=[pl.BlockSpec((NC, BR, DEGW), lambda i: (0, i, 0)),
                  pl.BlockSpec((BR, D), lambda i: (i, 0)),
                  pl.BlockSpec((D, D), lambda i: (0, 0))],
        out_specs=[pl.BlockSpec((BR, D), lambda i: (i, 0)),
                   pl.BlockSpec((BR, 1), lambda i: (i, 0))],
        out_shape=[jax.ShapeDtypeStruct((N, D), jnp.float32),
                   jax.ShapeDtypeStruct((N, 1), jnp.float32)],
    )(deg_parts, x, w)


def _tc_z_stats(s_parts, g, dis, b):
    def body(sp_ref, g_ref, dis_ref, b_ref, z_ref, st_ref):
        i = pl.program_id(0)
        z = dis_ref[...] * (sp_ref[0] + sp_ref[1] + g_ref[...]) + b_ref[...]
        z_ref[...] = z

        @pl.when(i == 0)
        def _():
            st_ref[...] = jnp.zeros_like(st_ref)

        st_ref[0:1, :] += jnp.sum(z, axis=0, keepdims=True)
        st_ref[1:2, :] += jnp.sum(z * z, axis=0, keepdims=True)

    return pl.pallas_call(
        body,
        grid=(N // BR,),
        in_specs=[pl.BlockSpec((NC, BR, D), lambda i: (0, i, 0)),
                  pl.BlockSpec((BR, D), lambda i: (i, 0)),
                  pl.BlockSpec((BR, 1), lambda i: (i, 0)),
                  pl.BlockSpec((1, D), lambda i: (0, 0))],
        out_specs=[pl.BlockSpec((BR, D), lambda i: (i, 0)),
                   pl.BlockSpec((2, D), lambda i: (0, 0))],
        out_shape=[jax.ShapeDtypeStruct((N, D), jnp.float32),
                   jax.ShapeDtypeStruct((2, D), jnp.float32)],
    )(s_parts, g, dis, b)


def _tc_bn_mm(z, st, dis, gamma, beta, w2):
    def body(z_ref, st_ref, dis_ref, ga_ref, be_ref, w_ref, o_ref):
        mean = st_ref[0:1, :] * (1.0 / N)
        var = st_ref[1:2, :] * (1.0 / N) - mean * mean
        inv = lax.rsqrt(var + 1e-5)
        r = (z_ref[...] - mean) * (inv * ga_ref[...]) + be_ref[...]
        r = jnp.maximum(r, 0.0)
        o_ref[...] = _dot(r, w_ref[...]) * dis_ref[...]

    return pl.pallas_call(
        body,
        grid=(N // BR,),
        in_specs=[pl.BlockSpec((BR, D), lambda i: (i, 0)),
                  pl.BlockSpec((2, D), lambda i: (0, 0)),
                  pl.BlockSpec((BR, 1), lambda i: (i, 0)),
                  pl.BlockSpec((1, D), lambda i: (0, 0)),
                  pl.BlockSpec((1, D), lambda i: (0, 0)),
                  pl.BlockSpec((D, D), lambda i: (0, 0))],
        out_specs=pl.BlockSpec((BR, D), lambda i: (i, 0)),
        out_shape=jax.ShapeDtypeStruct((N, D), jnp.float32),
    )(z, st, dis, gamma, beta, w2)


def _tc_combine(s_parts, g, dis, b):
    def body(sp_ref, g_ref, dis_ref, b_ref, o_ref):
        o_ref[...] = (dis_ref[...] * (sp_ref[0] + sp_ref[1] + g_ref[...])
                      + b_ref[...])

    return pl.pallas_call(
        body,
        grid=(N // BR,),
        in_specs=[pl.BlockSpec((NC, BR, D), lambda i: (0, i, 0)),
                  pl.BlockSpec((BR, D), lambda i: (i, 0)),
                  pl.BlockSpec((BR, 1), lambda i: (i, 0)),
                  pl.BlockSpec((1, D), lambda i: (0, 0))],
        out_specs=pl.BlockSpec((BR, D), lambda i: (i, 0)),
        out_shape=jax.ShapeDtypeStruct((N, D), jnp.float32),
    )(s_parts, g, dis, b)


def kernel(x, edge_index, W1, b1, gamma, beta, W2, b2):
    ei = edge_index.astype(jnp.int32)
    src = ei[0]
    dst = ei[1]

    deg_parts = _sc_degree(dst)
    g1, dis = _tc_mm_scale(deg_parts, x, W1)
    s1 = _sc_aggregate(g1, src, dst)
    z, st = _tc_z_stats(s1, g1, dis, b1.reshape(1, D))
    g2 = _tc_bn_mm(z, st, dis, gamma.reshape(1, D), beta.reshape(1, D), W2)
    s2 = _sc_aggregate(g2, src, dst)
    return _tc_combine(s2, g2, dis, b2.reshape(1, D))


# speedup vs baseline: 1.0139x; 1.0139x over previous
import functools

import jax
import jax.numpy as jnp
from jax import lax
from jax.experimental import pallas as pl
from jax.experimental.pallas import tpu as pltpu
from jax.experimental.pallas import tpu_sc as plsc

N = 10000
E = 320000
D = 128
NC = 2
NS = 16
NW = NC * NS
EPW = E // NW
CHUNK = 128
FULL_CHUNKS = EPW // CHUNK
TAIL = EPW - FULL_CHUNKS * CHUNK
RPS = 624
ZROWS = 16
LAST = NS * RPS
LREM = N - LAST
DEGW = 128
BR = 1000


def _vmesh():
    return plsc.VectorSubcoreMesh(core_axis_name="c", subcore_axis_name="s")


def _sc_degree(dst):
    DCH = 64
    DFC = (EPW - TAIL) // DCH
    DNIT = (DFC - 4) // 4

    @functools.partial(
        pl.kernel,
        out_type=jax.ShapeDtypeStruct((NC, N, DEGW), jnp.float32),
        mesh=_vmesh(),
        scratch_types=(
            [pltpu.VMEM((DCH,), jnp.int32)] * 4
            + [pltpu.VMEM((TAIL,), jnp.int32)]
            + [pltpu.VMEM((DCH, DEGW), jnp.float32)]
            + [pltpu.VMEM((ZROWS, DEGW), jnp.float32)]
            + [pltpu.VMEM_SHARED((N, DEGW), jnp.float32)]
            + [pltpu.SemaphoreType.DMA] * 8
        ),
    )
    def deg_kernel(dst_hbm, out_hbm, idx0, idx1, idx2, idx3, idxt_v, ones_v,
                   z_v, deg_sh, si0, si1, si2, si3, ss0, ss1, ss2, ss3):
        cid = lax.axis_index("c")
        sid = lax.axis_index("s")
        wid = cid * NS + sid
        one16 = jnp.ones((16,), jnp.float32)
        zero16 = jnp.zeros((16,), jnp.float32)
        base = wid * EPW

        bufs = [(idx0, si0, ss0), (idx1, si1, ss1),
                (idx2, si2, ss2), (idx3, si3, ss3)]

        def idx_start(c, b):
            pltpu.async_copy(dst_hbm.at[pl.ds(base + c * DCH, DCH)], b[0], b[1])

        def idx_wait(b):
            pltpu.make_async_copy(dst_hbm.at[pl.ds(base, DCH)], b[0], b[1]).wait()

        def scat_start(b):
            pltpu.async_copy(ones_v, deg_sh.at[b[0]], b[2], add=True)

        def scat_wait(b):
            pltpu.make_async_copy(ones_v, deg_sh.at[b[0]], b[2]).wait()

        idx_start(0, bufs[0])
        idx_start(1, bufs[1])

        @pl.loop(0, DCH)
        def _(i):
            @pl.loop(0, DEGW // 16)
            def _(j):
                ones_v[i, pl.ds(j * 16, 16)] = one16

        @pl.loop(0, ZROWS)
        def _(i):
            @pl.loop(0, DEGW // 16)
            def _(j):
                z_v[i, pl.ds(j * 16, 16)] = zero16

        @pl.loop(0, RPS // ZROWS)
        def _(k):
            pltpu.sync_copy(z_v, deg_sh.at[pl.ds(sid * RPS + k * ZROWS, ZROWS)])

        @pl.when(sid == NS - 1)
        def _():
            pltpu.sync_copy(z_v.at[pl.ds(0, LREM)], deg_sh.at[pl.ds(LAST, LREM)])

        plsc.subcore_barrier()

        idx_wait(bufs[0])
        scat_start(bufs[0])
        idx_start(2, bufs[2])
        idx_wait(bufs[1])
        scat_start(bufs[1])
        idx_start(3, bufs[3])
        idx_wait(bufs[2])
        scat_start(bufs[2])
        scat_wait(bufs[0])
        idx_start(4, bufs[0])
        idx_wait(bufs[3])
        scat_start(bufs[3])
        scat_wait(bufs[1])
        idx_start(5, bufs[1])

        @pl.loop(0, DNIT)
        def _(k):
            for p in range(4):
                c = 4 * k + 4 + p
                X = bufs[p]
                P = bufs[(p + 2) % 4]
                idx_wait(X)
                scat_start(X)
                scat_wait(P)
                if p <= 1:
                    idx_start(c + 2, P)
                else:
                    @pl.when(k < DNIT - 1)
                    def _(c=c, P=P):
                        idx_start(c + 2, P)

        scat_wait(bufs[(DFC - 2) % 4])
        scat_wait(bufs[(DFC - 1) % 4])

        pltpu.sync_copy(dst_hbm.at[pl.ds(base + DFC * DCH, TAIL)], idxt_v)
        pltpu.sync_copy(ones_v.at[pl.ds(0, TAIL)], deg_sh.at[idxt_v], add=True)

        plsc.subcore_barrier()
        pltpu.sync_copy(deg_sh.at[pl.ds(sid * RPS, RPS)],
                        out_hbm.at[cid].at[pl.ds(sid * RPS, RPS)])

        @pl.when(sid == NS - 1)
        def _():
            pltpu.sync_copy(deg_sh.at[pl.ds(LAST, LREM)],
                            out_hbm.at[cid].at[pl.ds(LAST, LREM)])

    return deg_kernel(dst)


def _sc_aggregate(g, src, dst):
    ACH = 64
    AFC = (EPW - TAIL) // ACH
    NB = 4
    NIT = (AFC - 3) // NB
    REM = (AFC - 3) - NB * NIT

    @functools.partial(
        pl.kernel,
        out_type=jax.ShapeDtypeStruct((NC, N, D), jnp.float32),
        mesh=_vmesh(),
        scratch_types=(
            [pltpu.VMEM((ACH,), jnp.int32)] * (2 * NB)
            + [pltpu.VMEM((TAIL,), jnp.int32)] * 2
            + [pltpu.VMEM((ACH, D), jnp.float32)] * NB
            + [pltpu.VMEM((TAIL, D), jnp.float32)]
            + [pltpu.VMEM((ZROWS, D), jnp.float32)]
            + [pltpu.VMEM_SHARED((N, D), jnp.float32)]
            + [pltpu.SemaphoreType.DMA] * (3 * NB)
        ),
    )
    def agg_kernel(g_hbm, src_hbm, dst_hbm, out_hbm,
                   si0, di0, si1, di1, si2, di2, si3, di3, sit_v, dit_v,
                   rows0, rows1, rows2, rows3, rowst_v, z_v, s_sh,
                   ssi0, sdi0, sg0, ssi1, sdi1, sg1,
                   ssi2, sdi2, sg2, ssi3, sdi3, sg3):
        cid = lax.axis_index("c")
        sid = lax.axis_index("s")
        wid = cid * NS + sid
        zero16 = jnp.zeros((16,), jnp.float32)

        bufs = [(si0, di0, rows0, ssi0, sdi0, sg0),
                (si1, di1, rows1, ssi1, sdi1, sg1),
                (si2, di2, rows2, ssi2, sdi2, sg2),
                (si3, di3, rows3, ssi3, sdi3, sg3)]

        @pl.loop(0, ZROWS)
        def _(i):
            @pl.loop(0, D // 16)
            def _(j):
                z_v[i, pl.ds(j * 16, 16)] = zero16

        @pl.loop(0, RPS // ZROWS)
        def _(k):
            pltpu.sync_copy(z_v, s_sh.at[pl.ds(sid * RPS + k * ZROWS, ZROWS)])

        @pl.when(sid == NS - 1)
        def _():
            pltpu.sync_copy(z_v.at[pl.ds(0, LREM)], s_sh.at[pl.ds(LAST, LREM)])

        plsc.subcore_barrier()
        base = wid * EPW

        def idx_load(c, b):
            off = base + c * ACH
            pltpu.async_copy(src_hbm.at[pl.ds(off, ACH)], b[0], b[3])
            pltpu.async_copy(dst_hbm.at[pl.ds(off, ACH)], b[1], b[4])
            pltpu.make_async_copy(src_hbm.at[pl.ds(base, ACH)], b[0], b[3]).wait()
            pltpu.make_async_copy(dst_hbm.at[pl.ds(base, ACH)], b[1], b[4]).wait()

        def gath_start(b):
            pltpu.async_copy(g_hbm.at[b[0]], b[2], b[5])

        def gath_wait(b):
            pltpu.make_async_copy(g_hbm.at[b[0]], b[2], b[5]).wait()

        def scat_start(b):
            pltpu.async_copy(b[2], s_sh.at[b[1]], b[5], add=True)

        def scat_wait(b):
            pltpu.make_async_copy(b[2], s_sh.at[b[1]], b[5]).wait()

        def step(c, X, W, start_next=True):
            gath_wait(X)
            scat_start(X)
            scat_wait(W)
            if start_next:
                idx_load(c + 3, W)
                gath_start(W)

        idx_load(0, bufs[0])
        gath_start(bufs[0])
        idx_load(1, bufs[1])
        gath_start(bufs[1])
        idx_load(2, bufs[2])
        gath_start(bufs[2])
        gath_wait(bufs[0])
        scat_start(bufs[0])
        idx_load(3, bufs[3])
        gath_start(bufs[3])
        step(1, bufs[1], bufs[0])
        step(2, bufs[2], bufs[1])

        @pl.loop(0, NIT)
        def _(k):
            for p in range(NB):
                c = NB * k + 3 + p
                X = bufs[(3 + p) % NB]
                W = bufs[(2 + p) % NB]
                if p <= 1:
                    step(c, X, W)
                else:
                    @pl.when(k < NIT - 1)
                    def _(c=c, X=X, W=W):
                        step(c, X, W)

                    @pl.when(k == NIT - 1)
                    def _(c=c, X=X, W=W):
                        step(c, X, W, start_next=False)

        cl = AFC - 1
        step(cl, bufs[cl % NB], bufs[(cl - 1) % NB], start_next=False)
        scat_wait(bufs[cl % NB])

        tbase = base + AFC * ACH
        pltpu.sync_copy(src_hbm.at[pl.ds(tbase, TAIL)], sit_v)
        pltpu.sync_copy(dst_hbm.at[pl.ds(tbase, TAIL)], dit_v)
        pltpu.sync_copy(g_hbm.at[sit_v], rowst_v)
        pltpu.sync_copy(rowst_v, s_sh.at[dit_v], add=True)

        plsc.subcore_barrier()
        pltpu.sync_copy(s_sh.at[pl.ds(sid * RPS, RPS)],
                        out_hbm.at[cid].at[pl.ds(sid * RPS, RPS)])

        @pl.when(sid == NS - 1)
        def _():
            pltpu.sync_copy(s_sh.at[pl.ds(LAST, LREM)],
                            out_hbm.at[cid].at[pl.ds(LAST, LREM)])

    return agg_kernel(g, src, dst)


def _dot(a, b):
    return lax.dot_general(a, b, (((1,), (0,)), ((), ())),
                           preferred_element_type=jnp.float32,
                           precision=lax.Precision.HIGHEST)


def _tc_matmul(x, w):
    def body(x_ref, w_ref, o_ref):
        o_ref[...] = _dot(x_ref[...], w_ref[...])

    return pl.pallas_call(
        body,
        grid=(N // BR,),
        in_specs=[pl.BlockSpec((BR, D), lambda i: (i, 0)),
                  pl.BlockSpec((D, D), lambda i: (0, 0))],
        out_specs=pl.BlockSpec((BR, D), lambda i: (i, 0)),
        out_shape=jax.ShapeDtypeStruct((N, D), jnp.float32),
    )(x, w)


def _tc_scale(deg_parts, h):
    def body(dp_ref, h_ref, g_ref, dis_ref):
        deg = dp_ref[0, :, 0:1] + dp_ref[1, :, 0:1] + 1.0
        dis = lax.rsqrt(deg)
        g_ref[...] = h_ref[...] * dis
        dis_ref[...] = dis

    return pl.pallas_call(
        body,
        grid=(N // BR,),
        in_specs=[pl.BlockSpec((NC, BR, DEGW), lambda i: (0, i, 0)),
                  pl.BlockSpec((BR, D), lambda i: (i, 0))],
        out_specs=[pl.BlockSpec((BR, D), lambda i: (i, 0)),
                   pl.BlockSpec((BR, 1), lambda i: (i, 0))],
        out_shape=[jax.ShapeDtypeStruct((N, D), jnp.float32),
                   jax.ShapeDtypeStruct((N, 1), jnp.float32)],
    )(deg_parts, h)


def _tc_z_stats(s_parts, g, dis, b):
    def body(sp_ref, g_ref, dis_ref, b_ref, z_ref, st_ref):
        i = pl.program_id(0)
        z = dis_ref[...] * (sp_ref[0] + sp_ref[1] + g_ref[...]) + b_ref[...]
        z_ref[...] = z

        @pl.when(i == 0)
        def _():
            st_ref[...] = jnp.zeros_like(st_ref)

        st_ref[0:1, :] += jnp.sum(z, axis=0, keepdims=True)
        st_ref[1:2, :] += jnp.sum(z * z, axis=0, keepdims=True)

    return pl.pallas_call(
        body,
        grid=(N // BR,),
        in_specs=[pl.BlockSpec((NC, BR, D), lambda i: (0, i, 0)),
                  pl.BlockSpec((BR, D), lambda i: (i, 0)),
                  pl.BlockSpec((BR, 1), lambda i: (i, 0)),
                  pl.BlockSpec((1, D), lambda i: (0, 0))],
        out_specs=[pl.BlockSpec((BR, D), lambda i: (i, 0)),
                   pl.BlockSpec((2, D), lambda i: (0, 0))],
        out_shape=[jax.ShapeDtypeStruct((N, D), jnp.float32),
                   jax.ShapeDtypeStruct((2, D), jnp.float32)],
    )(s_parts, g, dis, b)


def _tc_bn_mm(z, st, dis, gamma, beta, w2):
    def body(z_ref, st_ref, dis_ref, ga_ref, be_ref, w_ref, o_ref):
        mean = st_ref[0:1, :] * (1.0 / N)
        var = st_ref[1:2, :] * (1.0 / N) - mean * mean
        inv = lax.rsqrt(var + 1e-5)
        r = (z_ref[...] - mean) * (inv * ga_ref[...]) + be_ref[...]
        r = jnp.maximum(r, 0.0)
        o_ref[...] = _dot(r, w_ref[...]) * dis_ref[...]

    return pl.pallas_call(
        body,
        grid=(N // BR,),
        in_specs=[pl.BlockSpec((BR, D), lambda i: (i, 0)),
                  pl.BlockSpec((2, D), lambda i: (0, 0)),
                  pl.BlockSpec((BR, 1), lambda i: (i, 0)),
                  pl.BlockSpec((1, D), lambda i: (0, 0)),
                  pl.BlockSpec((1, D), lambda i: (0, 0)),
                  pl.BlockSpec((D, D), lambda i: (0, 0))],
        out_specs=pl.BlockSpec((BR, D), lambda i: (i, 0)),
        out_shape=jax.ShapeDtypeStruct((N, D), jnp.float32),
    )(z, st, dis, gamma, beta, w2)


def _tc_combine(s_parts, g, dis, b):
    def body(sp_ref, g_ref, dis_ref, b_ref, o_ref):
        o_ref[...] = (dis_ref[...] * (sp_ref[0] + sp_ref[1] + g_ref[...])
                      + b_ref[...])

    return pl.pallas_call(
        body,
        grid=(N // BR,),
        in_specs=[pl.BlockSpec((NC, BR, D), lambda i: (0, i, 0)),
                  pl.BlockSpec((BR, D), lambda i: (i, 0)),
                  pl.BlockSpec((BR, 1), lambda i: (i, 0)),
                  pl.BlockSpec((1, D), lambda i: (0, 0))],
        out_specs=pl.BlockSpec((BR, D), lambda i: (i, 0)),
        out_shape=jax.ShapeDtypeStruct((N, D), jnp.float32),
    )(s_parts, g, dis, b)


def kernel(x, edge_index, W1, b1, gamma, beta, W2, b2):
    ei = edge_index.astype(jnp.int32)
    src = ei[0]
    dst = ei[1]

    deg_parts = _sc_degree(dst)
    h1 = _tc_matmul(x, W1)
    g1, dis = _tc_scale(deg_parts, h1)
    s1 = _sc_aggregate(g1, src, dst)
    z, st = _tc_z_stats(s1, g1, dis, b1.reshape(1, D))
    g2 = _tc_bn_mm(z, st, dis, gamma.reshape(1, D), beta.reshape(1, D), W2)
    s2 = _sc_aggregate(g2, src, dst)
    return _tc_combine(s2, g2, dis, b2.reshape(1, D))


# speedup vs baseline: 1.0160x; 1.0020x over previous
import functools

import jax
import jax.numpy as jnp
from jax import lax
from jax.experimental import pallas as pl
from jax.experimental.pallas import tpu as pltpu
from jax.experimental.pallas import tpu_sc as plsc

N = 10000
E = 320000
D = 128
NC = 2
NS = 16
NW = NC * NS
EPW = E // NW
CHUNK = 128
FULL_CHUNKS = EPW // CHUNK
TAIL = EPW - FULL_CHUNKS * CHUNK
RPS = 624
ZROWS = 16
LAST = NS * RPS
LREM = N - LAST
DEGW = 128
BR = 1000


def _vmesh():
    return plsc.VectorSubcoreMesh(core_axis_name="c", subcore_axis_name="s")


def _sc_degree(dst):
    DCH = 64
    DFC = (EPW - TAIL) // DCH
    DNIT = (DFC - 4) // 4

    @functools.partial(
        pl.kernel,
        out_type=jax.ShapeDtypeStruct((NC, N, DEGW), jnp.float32),
        mesh=_vmesh(),
        scratch_types=(
            [pltpu.VMEM((DCH,), jnp.int32)] * 4
            + [pltpu.VMEM((TAIL,), jnp.int32)]
            + [pltpu.VMEM((DCH, DEGW), jnp.float32)]
            + [pltpu.VMEM((ZROWS, DEGW), jnp.float32)]
            + [pltpu.VMEM_SHARED((N, DEGW), jnp.float32)]
            + [pltpu.SemaphoreType.DMA] * 8
        ),
    )
    def deg_kernel(dst_hbm, out_hbm, idx0, idx1, idx2, idx3, idxt_v, ones_v,
                   z_v, deg_sh, si0, si1, si2, si3, ss0, ss1, ss2, ss3):
        cid = lax.axis_index("c")
        sid = lax.axis_index("s")
        wid = cid * NS + sid
        one16 = jnp.ones((16,), jnp.float32)
        zero16 = jnp.zeros((16,), jnp.float32)
        base = wid * EPW

        bufs = [(idx0, si0, ss0), (idx1, si1, ss1),
                (idx2, si2, ss2), (idx3, si3, ss3)]

        def idx_start(c, b):
            pltpu.async_copy(dst_hbm.at[pl.ds(base + c * DCH, DCH)], b[0], b[1])

        def idx_wait(b):
            pltpu.make_async_copy(dst_hbm.at[pl.ds(base, DCH)], b[0], b[1]).wait()

        def scat_start(b):
            pltpu.async_copy(ones_v, deg_sh.at[b[0]], b[2], add=True)

        def scat_wait(b):
            pltpu.make_async_copy(ones_v, deg_sh.at[b[0]], b[2]).wait()

        idx_start(0, bufs[0])
        idx_start(1, bufs[1])

        @pl.loop(0, DCH)
        def _(i):
            @pl.loop(0, DEGW // 16)
            def _(j):
                ones_v[i, pl.ds(j * 16, 16)] = one16

        @pl.loop(0, ZROWS)
        def _(i):
            @pl.loop(0, DEGW // 16)
            def _(j):
                z_v[i, pl.ds(j * 16, 16)] = zero16

        @pl.loop(0, RPS // ZROWS)
        def _(k):
            pltpu.sync_copy(z_v, deg_sh.at[pl.ds(sid * RPS + k * ZROWS, ZROWS)])

        @pl.when(sid == NS - 1)
        def _():
            pltpu.sync_copy(z_v.at[pl.ds(0, LREM)], deg_sh.at[pl.ds(LAST, LREM)])

        plsc.subcore_barrier()

        idx_wait(bufs[0])
        scat_start(bufs[0])
        idx_start(2, bufs[2])
        idx_wait(bufs[1])
        scat_start(bufs[1])
        idx_start(3, bufs[3])
        idx_wait(bufs[2])
        scat_start(bufs[2])
        scat_wait(bufs[0])
        idx_start(4, bufs[0])
        idx_wait(bufs[3])
        scat_start(bufs[3])
        scat_wait(bufs[1])
        idx_start(5, bufs[1])

        @pl.loop(0, DNIT)
        def _(k):
            for p in range(4):
                c = 4 * k + 4 + p
                X = bufs[p]
                P = bufs[(p + 2) % 4]
                idx_wait(X)
                scat_start(X)
                scat_wait(P)
                if p <= 1:
                    idx_start(c + 2, P)
                else:
                    @pl.when(k < DNIT - 1)
                    def _(c=c, P=P):
                        idx_start(c + 2, P)

        scat_wait(bufs[(DFC - 2) % 4])
        scat_wait(bufs[(DFC - 1) % 4])

        pltpu.sync_copy(dst_hbm.at[pl.ds(base + DFC * DCH, TAIL)], idxt_v)
        pltpu.sync_copy(ones_v.at[pl.ds(0, TAIL)], deg_sh.at[idxt_v], add=True)

        plsc.subcore_barrier()
        pltpu.sync_copy(deg_sh.at[pl.ds(sid * RPS, RPS)],
                        out_hbm.at[cid].at[pl.ds(sid * RPS, RPS)])

        @pl.when(sid == NS - 1)
        def _():
            pltpu.sync_copy(deg_sh.at[pl.ds(LAST, LREM)],
                            out_hbm.at[cid].at[pl.ds(LAST, LREM)])

    return deg_kernel(dst)


def _sc_aggregate(g, src, dst):
    ACH = 64
    AFC = (EPW - TAIL) // ACH
    NB = 5
    NPEEL = 6
    NIT = (AFC - NPEEL) // NB

    @functools.partial(
        pl.kernel,
        out_type=jax.ShapeDtypeStruct((NC, N, D), jnp.float32),
        mesh=_vmesh(),
        scratch_types=(
            [pltpu.VMEM((ACH,), jnp.int32)] * (2 * NB)
            + [pltpu.VMEM((TAIL,), jnp.int32)] * 2
            + [pltpu.VMEM((ACH, D), jnp.float32)] * NB
            + [pltpu.VMEM((TAIL, D), jnp.float32)]
            + [pltpu.VMEM((ZROWS, D), jnp.float32)]
            + [pltpu.VMEM_SHARED((N, D), jnp.float32)]
            + [pltpu.SemaphoreType.DMA] * (3 * NB)
        ),
    )
    def agg_kernel(g_hbm, src_hbm, dst_hbm, out_hbm,
                   si0, di0, si1, di1, si2, di2, si3, di3, si4, di4,
                   sit_v, dit_v,
                   rows0, rows1, rows2, rows3, rows4, rowst_v, z_v, s_sh,
                   ssi0, sdi0, sg0, ssi1, sdi1, sg1, ssi2, sdi2, sg2,
                   ssi3, sdi3, sg3, ssi4, sdi4, sg4):
        cid = lax.axis_index("c")
        sid = lax.axis_index("s")
        wid = cid * NS + sid
        zero16 = jnp.zeros((16,), jnp.float32)
        base = wid * EPW

        bufs = [(si0, di0, rows0, ssi0, sdi0, sg0),
                (si1, di1, rows1, ssi1, sdi1, sg1),
                (si2, di2, rows2, ssi2, sdi2, sg2),
                (si3, di3, rows3, ssi3, sdi3, sg3),
                (si4, di4, rows4, ssi4, sdi4, sg4)]

        def idx_load(c, b):
            off = base + c * ACH
            pltpu.async_copy(src_hbm.at[pl.ds(off, ACH)], b[0], b[3])
            pltpu.async_copy(dst_hbm.at[pl.ds(off, ACH)], b[1], b[4])
            pltpu.make_async_copy(src_hbm.at[pl.ds(base, ACH)], b[0], b[3]).wait()
            pltpu.make_async_copy(dst_hbm.at[pl.ds(base, ACH)], b[1], b[4]).wait()

        def gath_start(b):
            pltpu.async_copy(g_hbm.at[b[0]], b[2], b[5])

        def gath_wait(b):
            pltpu.make_async_copy(g_hbm.at[b[0]], b[2], b[5]).wait()

        def scat_start(b):
            pltpu.async_copy(b[2], s_sh.at[b[1]], b[5], add=True)

        def scat_wait(b):
            pltpu.make_async_copy(b[2], s_sh.at[b[1]], b[5]).wait()

        idx_load(0, bufs[0])
        gath_start(bufs[0])
        idx_load(1, bufs[1])
        gath_start(bufs[1])
        idx_load(2, bufs[2])
        gath_start(bufs[2])

        @pl.loop(0, ZROWS)
        def _(i):
            @pl.loop(0, D // 16)
            def _(j):
                z_v[i, pl.ds(j * 16, 16)] = zero16

        @pl.loop(0, RPS // ZROWS)
        def _(k):
            pltpu.sync_copy(z_v, s_sh.at[pl.ds(sid * RPS + k * ZROWS, ZROWS)])

        @pl.when(sid == NS - 1)
        def _():
            pltpu.sync_copy(z_v.at[pl.ds(0, LREM)], s_sh.at[pl.ds(LAST, LREM)])

        plsc.subcore_barrier()

        def step(c, X, W, retire=True, start_next=True):
            gath_wait(X)
            scat_start(X)
            if retire:
                scat_wait(W)
            if start_next:
                idx_load(c + 3, W)
                gath_start(W)

        step(0, bufs[0], bufs[3], retire=False)
        step(1, bufs[1], bufs[4], retire=False)
        step(2, bufs[2], bufs[0])
        step(3, bufs[3], bufs[1])
        step(4, bufs[4], bufs[2])
        step(5, bufs[0], bufs[3])

        @pl.loop(0, NIT)
        def _(k):
            for p in range(NB):
                c = NB * k + NPEEL + p
                X = bufs[(1 + p) % NB]
                W = bufs[(4 + p) % NB]
                if p <= 1:
                    step(c, X, W)
                else:
                    @pl.when(k < NIT - 1)
                    def _(c=c, X=X, W=W):
                        step(c, X, W)

                    @pl.when(k == NIT - 1)
                    def _(c=c, X=X, W=W):
                        step(c, X, W, start_next=False)

        scat_wait(bufs[(AFC - 2) % NB])
        scat_wait(bufs[(AFC - 1) % NB])

        tbase = base + AFC * ACH
        pltpu.sync_copy(src_hbm.at[pl.ds(tbase, TAIL)], sit_v)
        pltpu.sync_copy(dst_hbm.at[pl.ds(tbase, TAIL)], dit_v)
        pltpu.sync_copy(g_hbm.at[sit_v], rowst_v)
        pltpu.sync_copy(rowst_v, s_sh.at[dit_v], add=True)

        plsc.subcore_barrier()
        pltpu.sync_copy(s_sh.at[pl.ds(sid * RPS, RPS)],
                        out_hbm.at[cid].at[pl.ds(sid * RPS, RPS)])

        @pl.when(sid == NS - 1)
        def _():
            pltpu.sync_copy(s_sh.at[pl.ds(LAST, LREM)],
                            out_hbm.at[cid].at[pl.ds(LAST, LREM)])

    return agg_kernel(g, src, dst)


def _dot(a, b):
    return lax.dot_general(a, b, (((1,), (0,)), ((), ())),
                           preferred_element_type=jnp.float32,
                           precision=lax.Precision.HIGHEST)


def _tc_matmul(x, w):
    def body(x_ref, w_ref, o_ref):
        o_ref[...] = _dot(x_ref[...], w_ref[...])

    return pl.pallas_call(
        body,
        grid=(N // BR,),
        in_specs=[pl.BlockSpec((BR, D), lambda i: (i, 0)),
                  pl.BlockSpec((D, D), lambda i: (0, 0))],
        out_specs=pl.BlockSpec((BR, D), lambda i: (i, 0)),
        out_shape=jax.ShapeDtypeStruct((N, D), jnp.float32),
    )(x, w)


def _tc_scale(deg_parts, h):
    def body(dp_ref, h_ref, g_ref, dis_ref):
        deg = dp_ref[0, :, 0:1] + dp_ref[1, :, 0:1] + 1.0
        dis = lax.rsqrt(deg)
        g_ref[...] = h_ref[...] * dis
        dis_ref[...] = dis

    return pl.pallas_call(
        body,
        grid=(N // BR,),
        in_specs=[pl.BlockSpec((NC, BR, DEGW), lambda i: (0, i, 0)),
                  pl.BlockSpec((BR, D), lambda i: (i, 0))],
        out_specs=[pl.BlockSpec((BR, D), lambda i: (i, 0)),
                   pl.BlockSpec((BR, 1), lambda i: (i, 0))],
        out_shape=[jax.ShapeDtypeStruct((N, D), jnp.float32),
                   jax.ShapeDtypeStruct((N, 1), jnp.float32)],
    )(deg_parts, h)


def _tc_z_stats(s_parts, g, dis, b):
    def body(sp_ref, g_ref, dis_ref, b_ref, z_ref, st_ref):
        i = pl.program_id(0)
        z = dis_ref[...] * (sp_ref[0] + sp_ref[1] + g_ref[...]) + b_ref[...]
        z_ref[...] = z

        @pl.when(i == 0)
        def _():
            st_ref[...] = jnp.zeros_like(st_ref)

        st_ref[0:1, :] += jnp.sum(z, axis=0, keepdims=True)
        st_ref[1:2, :] += jnp.sum(z * z, axis=0, keepdims=True)

    return pl.pallas_call(
        body,
        grid=(N // BR,),
        in_specs=[pl.BlockSpec((NC, BR, D), lambda i: (0, i, 0)),
                  pl.BlockSpec((BR, D), lambda i: (i, 0)),
                  pl.BlockSpec((BR, 1), lambda i: (i, 0)),
                  pl.BlockSpec((1, D), lambda i: (0, 0))],
        out_specs=[pl.BlockSpec((BR, D), lambda i: (i, 0)),
                   pl.BlockSpec((2, D), lambda i: (0, 0))],
        out_shape=[jax.ShapeDtypeStruct((N, D), jnp.float32),
                   jax.ShapeDtypeStruct((2, D), jnp.float32)],
    )(s_parts, g, dis, b)


def _tc_bn_mm(z, st, dis, gamma, beta, w2):
    def body(z_ref, st_ref, dis_ref, ga_ref, be_ref, w_ref, o_ref):
        mean = st_ref[0:1, :] * (1.0 / N)
        var = st_ref[1:2, :] * (1.0 / N) - mean * mean
        inv = lax.rsqrt(var + 1e-5)
        r = (z_ref[...] - mean) * (inv * ga_ref[...]) + be_ref[...]
        r = jnp.maximum(r, 0.0)
        o_ref[...] = _dot(r, w_ref[...]) * dis_ref[...]

    return pl.pallas_call(
        body,
        grid=(N // BR,),
        in_specs=[pl.BlockSpec((BR, D), lambda i: (i, 0)),
                  pl.BlockSpec((2, D), lambda i: (0, 0)),
                  pl.BlockSpec((BR, 1), lambda i: (i, 0)),
                  pl.BlockSpec((1, D), lambda i: (0, 0)),
                  pl.BlockSpec((1, D), lambda i: (0, 0)),
                  pl.BlockSpec((D, D), lambda i: (0, 0))],
        out_specs=pl.BlockSpec((BR, D), lambda i: (i, 0)),
        out_shape=jax.ShapeDtypeStruct((N, D), jnp.float32),
    )(z, st, dis, gamma, beta, w2)


def _tc_combine(s_parts, g, dis, b):
    def body(sp_ref, g_ref, dis_ref, b_ref, o_ref):
        o_ref[...] = (dis_ref[...] * (sp_ref[0] + sp_ref[1] + g_ref[...])
                      + b_ref[...])

    return pl.pallas_call(
        body,
        grid=(N // BR,),
        in_specs=[pl.BlockSpec((NC, BR, D), lambda i: (0, i, 0)),
                  pl.BlockSpec((BR, D), lambda i: (i, 0)),
                  pl.BlockSpec((BR, 1), lambda i: (i, 0)),
                  pl.BlockSpec((1, D), lambda i: (0, 0))],
        out_specs=pl.BlockSpec((BR, D), lambda i: (i, 0)),
        out_shape=jax.ShapeDtypeStruct((N, D), jnp.float32),
    )(s_parts, g, dis, b)


def kernel(x, edge_index, W1, b1, gamma, beta, W2, b2):
    ei = edge_index.astype(jnp.int32)
    src = ei[0]
    dst = ei[1]

    deg_parts = _sc_degree(dst)
    h1 = _tc_matmul(x, W1)
    g1, dis = _tc_scale(deg_parts, h1)
    s1 = _sc_aggregate(g1, src, dst)
    z, st = _tc_z_stats(s1, g1, dis, b1.reshape(1, D))
    g2 = _tc_bn_mm(z, st, dis, gamma.reshape(1, D), beta.reshape(1, D), W2)
    s2 = _sc_aggregate(g2, src, dst)
    return _tc_combine(s2, g2, dis, b2.reshape(1, D))
